# baseline (speedup 1.0000x reference)
import math

import jax
import jax.numpy as jnp
from jax import lax
from jax.experimental import pallas as pl
from jax.experimental.pallas import tpu as pltpu
from jax.experimental.pallas import tpu_sc as plsc

D_MODEL = 128
SCALE = math.sqrt(D_MODEL)
NC, NS, LANES = 2, 16, 16
NW = NC * NS
CHUNK = 128


def _body(x3d_hbm, table_hbm, out_hbm, idx_v, gin, gout, gsem, ssem):
    n_chunks = x3d_hbm.shape[1]
    n_outer = n_chunks // 2
    wid = lax.axis_index("s") * NC + lax.axis_index("c")
    base = wid * n_chunks
    pltpu.sync_copy(x3d_hbm.at[wid], idx_v)

    for b in range(2):
        pltpu.async_copy(table_hbm.at[idx_v.at[b]], gin.at[b], gsem.at[b])

    def outer(step, carry):
        for b in range(2):
            c = step * 2 + b
            gin_b = gin.at[b]
            gout_b = gout.at[b]
            pltpu.make_async_copy(table_hbm.at[idx_v.at[c]], gin_b,
                                  gsem.at[b]).wait()
            @pl.when(step >= 1)
            def _():
                pltpu.make_async_copy(
                    gout_b, out_hbm.at[pl.ds(0, CHUNK)], ssem.at[b]).wait()

            def scale_row(r, carry2):
                for j in range(D_MODEL // LANES):
                    sl = pl.ds(j * LANES, LANES)
                    gout_b[r, sl] = gin_b[r, sl] * SCALE
                return carry2

            lax.fori_loop(0, CHUNK, scale_row, 0, unroll=2)
            pltpu.async_copy(
                gout_b, out_hbm.at[pl.ds((base + c) * CHUNK, CHUNK)],
                ssem.at[b])
            @pl.when(step < n_outer - 1)
            def _():
                pltpu.async_copy(table_hbm.at[idx_v.at[c + 2]], gin_b,
                                 gsem.at[b])
        return carry

    lax.fori_loop(0, n_outer, outer, 0)
    for b in range(2):
        pltpu.make_async_copy(gout.at[b], out_hbm.at[pl.ds(0, CHUNK)],
                              ssem.at[b]).wait()


def kernel(x, table):
    orig_shape = x.shape
    b_total = x.size
    assert b_total % (NW * CHUNK) == 0
    n_chunks = b_total // (NW * CHUNK)
    x3d = x.reshape(NW, n_chunks, CHUNK).astype(jnp.int32)

    mesh = plsc.VectorSubcoreMesh(core_axis_name="c", subcore_axis_name="s")
    out = pl.kernel(
        _body,
        out_type=jax.ShapeDtypeStruct((b_total, D_MODEL), jnp.float32),
        mesh=mesh,
        scratch_types=[
            pltpu.VMEM((n_chunks, CHUNK), jnp.int32),
            pltpu.VMEM((2, CHUNK, D_MODEL), jnp.float32),
            pltpu.VMEM((2, CHUNK, D_MODEL), jnp.float32),
            pltpu.SemaphoreType.DMA((2,)),
            pltpu.SemaphoreType.DMA((2,)),
        ],
    )(x3d, table)
    return out.reshape(*orig_shape, D_MODEL)


# speedup vs baseline: 1.1754x; 1.1754x over previous
import math

import jax
import jax.numpy as jnp
from jax import lax
from jax.experimental import pallas as pl
from jax.experimental.pallas import tpu as pltpu
from jax.experimental.pallas import tpu_sc as plsc

D_MODEL = 128
SCALE = math.sqrt(D_MODEL)
NC, NS, LANES = 2, 16, 16
NW = NC * NS


def _body(x_hbm, table_hbm, out_hbm, idx_v, gin, gout, gsem, ssem):
    n_rows = x_hbm.shape[0]
    seq = x_hbm.shape[1]
    rows_per_w = n_rows // NW
    wid = lax.axis_index("s") * NC + lax.axis_index("c")
    base = wid * rows_per_w
    pltpu.sync_copy(x_hbm.at[pl.ds(base, rows_per_w)], idx_v)

    for b in range(2):
        pltpu.async_copy(table_hbm.at[idx_v.at[b]], gin.at[b], gsem.at[b])

    n_outer = rows_per_w // 2

    def outer(step, carry):
        for b in range(2):
            r = step * 2 + b
            gin_b = gin.at[b]
            gout_b = gout.at[b]
            pltpu.make_async_copy(table_hbm.at[idx_v.at[r]], gin_b,
                                  gsem.at[b]).wait()
            @pl.when(step >= 1)
            def _():
                pltpu.make_async_copy(gout_b, out_hbm.at[0], ssem.at[b]).wait()

            def scale_row(i, carry2):
                for j in range(D_MODEL // LANES):
                    sl = pl.ds(j * LANES, LANES)
                    gout_b[i, sl] = gin_b[i, sl] * SCALE
                return carry2

            lax.fori_loop(0, seq, scale_row, 0, unroll=2)
            pltpu.async_copy(gout_b, out_hbm.at[base + r], ssem.at[b])
            @pl.when(step < n_outer - 1)
            def _():
                pltpu.async_copy(table_hbm.at[idx_v.at[r + 2]], gin_b,
                                 gsem.at[b])
        return carry

    lax.fori_loop(0, n_outer, outer, 0)
    for b in range(2):
        pltpu.make_async_copy(gout.at[b], out_hbm.at[0], ssem.at[b]).wait()


def kernel(x, table):
    n_rows, seq = x.shape
    mesh = plsc.VectorSubcoreMesh(core_axis_name="c", subcore_axis_name="s")
    out = pl.kernel(
        _body,
        out_type=jax.ShapeDtypeStruct((n_rows, seq, D_MODEL), jnp.float32),
        mesh=mesh,
        compiler_params=pltpu.CompilerParams(use_tc_tiling_on_sc=True),
        scratch_types=[
            pltpu.VMEM((n_rows // NW, seq), jnp.int32),
            pltpu.VMEM((2, seq, D_MODEL), jnp.float32),
            pltpu.VMEM((2, seq, D_MODEL), jnp.float32),
            pltpu.SemaphoreType.DMA((2,)),
            pltpu.SemaphoreType.DMA((2,)),
        ],
    )(x, table)
    return out


# speedup vs baseline: 1.9336x; 1.6451x over previous
import math

import jax
import jax.numpy as jnp
from jax import lax
from jax.experimental import pallas as pl
from jax.experimental.pallas import tpu as pltpu
from jax.experimental.pallas import tpu_sc as plsc

D_MODEL = 128
SCALE = math.sqrt(D_MODEL)
NC, NS, LANES = 2, 16, 16
NW = NC * NS


def _body(x_hbm, table_hbm, out_hbm, idx_v, gin, gout, gsem, ssem):
    n_rows = x_hbm.shape[0]
    seq = x_hbm.shape[1]
    rows_per_w = n_rows // NW
    wid = lax.axis_index("s") * NC + lax.axis_index("c")
    base = wid * rows_per_w
    pltpu.sync_copy(x_hbm.at[pl.ds(base, rows_per_w)], idx_v)

    for b in range(2):
        pltpu.async_copy(table_hbm.at[idx_v.at[b]], gin.at[b], gsem.at[b])

    n_outer = rows_per_w // 2

    def outer(step, carry):
        for b in range(2):
            r = step * 2 + b
            gin_b = gin.at[b]
            gout_b = gout.at[b]
            pltpu.make_async_copy(table_hbm.at[idx_v.at[r]], gin_b,
                                  gsem.at[b]).wait()
            @pl.when(step >= 1)
            def _():
                pltpu.make_async_copy(gout_b, out_hbm.at[0], ssem.at[b]).wait()

            def scale_row(i, carry2):
                vals = [gin_b[i, pl.ds(j * LANES, LANES)]
                        for j in range(D_MODEL // LANES)]
                for j in range(D_MODEL // LANES):
                    gout_b[i, pl.ds(j * LANES, LANES)] = vals[j] * SCALE
                return carry2

            lax.fori_loop(0, seq, scale_row, 0, unroll=2)
            pltpu.async_copy(gout_b, out_hbm.at[base + r], ssem.at[b])
            @pl.when(step < n_outer - 1)
            def _():
                pltpu.async_copy(table_hbm.at[idx_v.at[r + 2]], gin_b,
                                 gsem.at[b])
        return carry

    lax.fori_loop(0, n_outer, outer, 0)
    for b in range(2):
        pltpu.make_async_copy(gout.at[b], out_hbm.at[0], ssem.at[b]).wait()


def kernel(x, table):
    n_rows, seq = x.shape
    mesh = plsc.VectorSubcoreMesh(core_axis_name="c", subcore_axis_name="s")
    out = pl.kernel(
        _body,
        out_type=jax.ShapeDtypeStruct((n_rows, seq, D_MODEL), jnp.float32),
        mesh=mesh,
        compiler_params=pltpu.CompilerParams(use_tc_tiling_on_sc=True),
        scratch_types=[
            pltpu.VMEM((n_rows // NW, seq), jnp.int32),
            pltpu.VMEM((2, seq, D_MODEL), jnp.float32),
            pltpu.VMEM((2, seq, D_MODEL), jnp.float32),
            pltpu.SemaphoreType.DMA((2,)),
            pltpu.SemaphoreType.DMA((2,)),
        ],
    )(x, table)
    return out


# speedup vs baseline: 2.3298x; 1.2049x over previous
import math

import jax
import jax.numpy as jnp
from jax import lax
from jax.experimental import pallas as pl
from jax.experimental.pallas import tpu as pltpu
from jax.experimental.pallas import tpu_sc as plsc

D_MODEL = 128
SCALE = math.sqrt(D_MODEL)
NC, NS, LANES = 2, 16, 16
NW = NC * NS
G = 4


def _body(x_hbm, table_hbm, out_hbm, idx_v, gin, gout, gsem, ssem):
    n_rows = x_hbm.shape[0]
    seq = x_hbm.shape[1]
    rows_per_w = n_rows // NW
    wid = lax.axis_index("s") * NC + lax.axis_index("c")
    base = wid * rows_per_w
    pltpu.sync_copy(x_hbm.at[pl.ds(base, rows_per_w)], idx_v)

    def start_gathers(step, b):
        for g in range(G):
            pltpu.async_copy(table_hbm.at[idx_v.at[step * G + g]],
                             gin.at[b, g], gsem.at[b])

    def wait_gathers(step, b):
        for g in range(G):
            pltpu.make_async_copy(table_hbm.at[idx_v.at[step * G + g]],
                                  gin.at[b, g], gsem.at[b]).wait()

    for b in range(2):
        start_gathers(b, b)

    n_steps = rows_per_w // G

    def outer(hstep, carry):
        for b in range(2):
            step = hstep * 2 + b
            gin_b = gin.at[b]
            gout_b = gout.at[b]
            wait_gathers(step, b)
            @pl.when(hstep >= 1)
            def _():
                pltpu.make_async_copy(gout_b, out_hbm.at[pl.ds(0, G)],
                                      ssem.at[b]).wait()

            def scale_row(i, carry2):
                vals = [gin_b[g, i, pl.ds(j * LANES, LANES)]
                        for g in range(G) for j in range(D_MODEL // LANES)]
                k = 0
                for g in range(G):
                    for j in range(D_MODEL // LANES):
                        gout_b[g, i, pl.ds(j * LANES, LANES)] = \
                            vals[k] * SCALE
                        k += 1
                return carry2

            lax.fori_loop(0, seq, scale_row, 0)
            pltpu.async_copy(gout_b, out_hbm.at[pl.ds(base + step * G, G)],
                             ssem.at[b])
            @pl.when(hstep < n_steps // 2 - 1)
            def _():
                start_gathers(step + 2, b)
        return carry

    lax.fori_loop(0, n_steps // 2, outer, 0)
    for b in range(2):
        pltpu.make_async_copy(gout.at[b], out_hbm.at[pl.ds(0, G)],
                              ssem.at[b]).wait()


def kernel(x, table):
    n_rows, seq = x.shape
    mesh = plsc.VectorSubcoreMesh(core_axis_name="c", subcore_axis_name="s")
    out = pl.kernel(
        _body,
        out_type=jax.ShapeDtypeStruct((n_rows, seq, D_MODEL), jnp.float32),
        mesh=mesh,
        compiler_params=pltpu.CompilerParams(use_tc_tiling_on_sc=True),
        scratch_types=[
            pltpu.VMEM((n_rows // NW, seq), jnp.int32),
            pltpu.VMEM((2, G, seq, D_MODEL), jnp.float32),
            pltpu.VMEM((2, G, seq, D_MODEL), jnp.float32),
            pltpu.SemaphoreType.DMA((2,)),
            pltpu.SemaphoreType.DMA((2,)),
        ],
    )(x, table)
    return out
